# Initial kernel scaffold; baseline (speedup 1.0000x reference)
import jax
import jax.numpy as jnp
from jax.experimental import pallas as pl


def kernel(x, edge_index, W1, b1, W2, b2, W3, b3):
    raise NotImplementedError("write your pallas kernel here")


# speedup vs baseline: 2.1681x; 2.1681x over previous
import functools

import jax
import jax.numpy as jnp
from jax.experimental import pallas as pl


def _dense_body(a_ref, w_ref, b_ref, o_ref):
    a = a_ref[...]
    w = w_ref[...]
    out = jnp.dot(a, w, preferred_element_type=jnp.float32)
    o_ref[...] = jax.nn.relu(out + b_ref[...])


@functools.partial(jax.jit, static_argnames=())
def _dense(a, w, b):
    n, _ = a.shape
    d_out = w.shape[1]
    return pl.pallas_call(
        _dense_body,
        out_shape=jax.ShapeDtypeStruct((n, d_out), jnp.float32),
    )(a, w, b.reshape(1, d_out))


def kernel(x, edge_index, W1, b1, W2, b2, W3, b3):
    n = x.shape[0]
    src, dst = edge_index[0], edge_index[1]

    ones = jnp.ones(src.shape, jnp.float32)
    indeg = jnp.zeros((n,), jnp.float32).at[dst].add(ones)
    dinv = jax.lax.rsqrt(indeg + 1.0)

    h = x
    for W, b in ((W1, b1), (W2, b2), (W3, b3)):
        y = h * dinv[:, None]
        seg = jnp.zeros(y.shape, jnp.float32).at[dst].add(jnp.take(y, src, axis=0))
        agg = (seg + y) * dinv[:, None]
        h = _dense(agg, W, b)
        m = jax.ops.segment_max(jnp.take(h, src, axis=0), dst, num_segments=n)
        h = jnp.maximum(m, h)
    return h


# speedup vs baseline: 3.4517x; 1.5920x over previous
import functools

import jax
import jax.numpy as jnp
from jax import lax
from jax.experimental import pallas as pl
from jax.experimental.pallas import tpu as pltpu
from jax.experimental.pallas import tpu_sc as plsc

N = 10000
E = 320000
NC, NS = 2, 16
NW = NC * NS
EPW = E // NW
ROWS_PER_SUB = 632
NPAD = ROWS_PER_SUB * NS

_MESH = plsc.VectorSubcoreMesh(core_axis_name="c", subcore_axis_name="s")


def _seg_sum(y, src, dst):
    F = y.shape[1]
    assert F <= 64
    K = 1000

    @functools.partial(
        pl.kernel,
        out_type=jax.ShapeDtypeStruct((NC * NPAD, F), jnp.float32),
        mesh=_MESH,
        compiler_params=pltpu.CompilerParams(use_tc_tiling_on_sc=False),
        scratch_types=[
            pltpu.VMEM((K,), jnp.int32),
            pltpu.VMEM((K,), jnp.int32),
            pltpu.VMEM((K, F), jnp.float32),
            pltpu.VMEM_SHARED((NPAD, F), jnp.float32),
            pltpu.SemaphoreType.DMA,
        ],
    )
    def ksum(y_hbm, src_hbm, dst_hbm, zero_hbm, out_hbm, sidx, didx, rows, acc, sem):
        cid = lax.axis_index("c")
        sid = lax.axis_index("s")
        wid = sid * NC + cid
        pltpu.sync_copy(
            zero_hbm.at[pl.ds(sid * ROWS_PER_SUB, ROWS_PER_SUB)],
            acc.at[pl.ds(sid * ROWS_PER_SUB, ROWS_PER_SUB)],
        )
        plsc.subcore_barrier()
        base = wid * EPW

        @pl.loop(0, EPW, step=K)
        def _(i):
            pltpu.sync_copy(src_hbm.at[pl.ds(base + i, K)], sidx)
            pltpu.sync_copy(dst_hbm.at[pl.ds(base + i, K)], didx)
            pltpu.async_copy(y_hbm.at[sidx], rows, sem).wait()
            pltpu.sync_copy(rows, acc.at[didx], add=True)

        plsc.subcore_barrier()
        pltpu.sync_copy(
            acc.at[pl.ds(sid * ROWS_PER_SUB, ROWS_PER_SUB)],
            out_hbm.at[pl.ds(cid * NPAD + sid * ROWS_PER_SUB, ROWS_PER_SUB)],
        )

    zeros = jnp.zeros((NPAD, F), jnp.float32)
    return ksum(y, src, dst, zeros)


def _dense_body(a_ref, w_ref, b_ref, o_ref):
    out = jnp.dot(a_ref[...], w_ref[...], preferred_element_type=jnp.float32)
    o_ref[...] = jax.nn.relu(out + b_ref[...])


def _dense(a, w, b):
    n, _ = a.shape
    d_out = w.shape[1]
    return pl.pallas_call(
        _dense_body,
        out_shape=jax.ShapeDtypeStruct((n, d_out), jnp.float32),
    )(a, w, b.reshape(1, d_out))


def kernel(x, edge_index, W1, b1, W2, b2, W3, b3):
    src, dst = edge_index[0], edge_index[1]

    deg_parts = _seg_sum(jnp.ones((N, 16), jnp.float32), src, dst)
    deg = deg_parts[:N, 0] + deg_parts[NPAD:NPAD + N, 0]
    dinv = lax.rsqrt(deg + 1.0)

    h = x
    for W, b in ((W1, b1), (W2, b2), (W3, b3)):
        f = h.shape[1]
        y = h * dinv[:, None]
        yp = jnp.pad(y, ((0, 0), (0, 16 - f))) if f < 16 else y
        if f > 64:
            halves = []
            for j in range(0, f, 64):
                p = _seg_sum(yp[:, j:j + 64], src, dst)
                halves.append(p[:N] + p[NPAD:NPAD + N])
            seg = jnp.concatenate(halves, axis=1)
        else:
            parts = _seg_sum(yp, src, dst)
            seg = (parts[:N, :f] + parts[NPAD:NPAD + N, :f])
        agg = (seg + y) * dinv[:, None]
        h = _dense(agg, W, b)
        m = jax.ops.segment_max(jnp.take(h, src, axis=0), dst, num_segments=N)
        h = jnp.maximum(m, h)
    return h


# speedup vs baseline: 12.8481x; 3.7222x over previous
import functools

import jax
import jax.numpy as jnp
from jax import lax
from jax.experimental import pallas as pl
from jax.experimental.pallas import tpu as pltpu
from jax.experimental.pallas import tpu_sc as plsc

N = 10000
E = 320000
NC, NS = 2, 16
NW = NC * NS
EPW = E // NW
ROWS_PER_SUB = 632
NPAD = ROWS_PER_SUB * NS

_MESH = plsc.VectorSubcoreMesh(core_axis_name="c", subcore_axis_name="s")


def _seg_sum(y, src, dst):
    F = y.shape[1]
    assert F <= 64
    K = 1000

    @functools.partial(
        pl.kernel,
        out_type=jax.ShapeDtypeStruct((NC * NPAD, F), jnp.float32),
        mesh=_MESH,
        compiler_params=pltpu.CompilerParams(use_tc_tiling_on_sc=False),
        scratch_types=[
            pltpu.VMEM((K,), jnp.int32),
            pltpu.VMEM((K,), jnp.int32),
            pltpu.VMEM((K, F), jnp.float32),
            pltpu.VMEM_SHARED((NPAD, F), jnp.float32),
            pltpu.SemaphoreType.DMA,
        ],
    )
    def ksum(y_hbm, src_hbm, dst_hbm, zero_hbm, out_hbm, sidx, didx, rows, acc, sem):
        cid = lax.axis_index("c")
        sid = lax.axis_index("s")
        wid = sid * NC + cid
        pltpu.sync_copy(
            zero_hbm.at[pl.ds(sid * ROWS_PER_SUB, ROWS_PER_SUB)],
            acc.at[pl.ds(sid * ROWS_PER_SUB, ROWS_PER_SUB)],
        )
        plsc.subcore_barrier()
        base = wid * EPW

        @pl.loop(0, EPW, step=K)
        def _(i):
            pltpu.sync_copy(src_hbm.at[pl.ds(base + i, K)], sidx)
            pltpu.sync_copy(dst_hbm.at[pl.ds(base + i, K)], didx)
            pltpu.async_copy(y_hbm.at[sidx], rows, sem).wait()
            pltpu.sync_copy(rows, acc.at[didx], add=True)

        plsc.subcore_barrier()
        pltpu.sync_copy(
            acc.at[pl.ds(sid * ROWS_PER_SUB, ROWS_PER_SUB)],
            out_hbm.at[pl.ds(cid * NPAD + sid * ROWS_PER_SUB, ROWS_PER_SUB)],
        )

    zeros = jnp.zeros((NPAD, F), jnp.float32)
    return ksum(y, src, dst, zeros)


NODES_PER_SUB = 320
NSEG = NODES_PER_SUB * NW
_KE = 512


def _seg_max(tables, rp_pad, srt_pad, dst_pad):
    nt = len(tables)
    FC = tables[0].shape[1]
    nfc = FC // 16
    RW = 336

    @functools.partial(
        pl.kernel,
        out_type=jax.ShapeDtypeStruct((nt * NSEG * FC,), jnp.float32),
        mesh=_MESH,
        compiler_params=pltpu.CompilerParams(use_tc_tiling_on_sc=False),
        scratch_types=[
            pltpu.VMEM((RW,), jnp.int32),
            pltpu.VMEM((_KE,), jnp.int32),
            pltpu.VMEM((_KE,), jnp.int32),
            pltpu.VMEM((_KE, FC), jnp.float32),
            pltpu.VMEM(((NODES_PER_SUB + 1) * FC,), jnp.float32),
            pltpu.SemaphoreType.DMA,
        ],
    )
    def kmax(*refs):
        (rp_hbm, srt_hbm, dst_hbm), tabs, scratch = (
            refs[:3], refs[3:3 + nt], refs[3 + nt:])
        out_hbm, rpv, sidx, dstv, rows, obuf, sem = scratch
        cid = lax.axis_index("c")
        sid = lax.axis_index("s")
        wid = sid * NC + cid
        lo = wid * NODES_PER_SUB
        pltpu.sync_copy(rp_hbm.at[pl.ds(lo, RW)], rpv)
        S = rpv[pl.ds(0, 16)][0]
        T = rpv[pl.ds(NODES_PER_SUB, 16)][0]
        S16 = pl.multiple_of((S // 16) * 16, 8)
        T16 = ((T + 15) // 16) * 16
        nch = (T16 - S16 + _KE - 1) // _KE
        zero = jnp.zeros((16,), jnp.float32)

        def rsafe(d):
            r = d - lo
            ok = (r >= 0) & (r < NODES_PER_SUB)
            return jnp.where(ok, r, NODES_PER_SUB)

        for t in range(nt):
            @pl.loop(0, NODES_PER_SUB + 1)
            def _(i):
                for k in range(nfc):
                    obuf[pl.ds(i * FC + 16 * k, 16)] = zero

            def do_chunk(c, carry):
                lo_c = pl.multiple_of(S16 + c * _KE, 8)
                ng = (jnp.minimum(lo_c + _KE, T16) - lo_c) // 16
                pltpu.sync_copy(srt_hbm.at[pl.ds(lo_c, _KE)], sidx)
                pltpu.sync_copy(dst_hbm.at[pl.ds(lo_c, _KE)], dstv)
                pltpu.async_copy(tabs[t].at[sidx], rows, sem).wait()

                def do_group(gi, st):
                    base = gi * 16
                    dvec = dstv[pl.ds(base, 16)]
                    for l in range(16):
                        pd, acc = st[0], st[1:]
                        d = dvec[l]
                        neq = d != pd
                        rowk = tuple(rows[base + l, pl.ds(16 * k, 16)]
                                     for k in range(nfc))

                        def flush(pd=pd, acc=acc):
                            ro = rsafe(pd) * FC
                            for k in range(nfc):
                                obuf[pl.ds(ro + 16 * k, 16)] = acc[k]

                        lax.cond(neq, flush, lambda: None)
                        st = (d,) + tuple(
                            jnp.maximum(jnp.where(neq, zero, acc[k]), rowk[k])
                            for k in range(nfc))
                    return st

                return lax.fori_loop(0, ng, do_group, carry)

            st = lax.fori_loop(0, nch, do_chunk,
                               (jnp.int32(-1),) + (zero,) * nfc)
            pd, acc = st[0], st[1:]
            ro = rsafe(pd) * FC
            for k in range(nfc):
                obuf[pl.ds(ro + 16 * k, 16)] = acc[k]
            pltpu.sync_copy(
                obuf.at[pl.ds(0, NODES_PER_SUB * FC)],
                out_hbm.at[pl.ds((t * NSEG + lo) * FC, NODES_PER_SUB * FC)])

    return kmax(rp_pad, srt_pad, dst_pad, *tables).reshape(nt * NSEG, FC)


def _dense_body(a_ref, w_ref, b_ref, o_ref):
    out = jnp.dot(a_ref[...], w_ref[...], preferred_element_type=jnp.float32)
    o_ref[...] = jax.nn.relu(out + b_ref[...])


def _dense(a, w, b):
    n, _ = a.shape
    d_out = w.shape[1]
    return pl.pallas_call(
        _dense_body,
        out_shape=jax.ShapeDtypeStruct((n, d_out), jnp.float32),
    )(a, w, b.reshape(1, d_out))


def kernel(x, edge_index, W1, b1, W2, b2, W3, b3):
    src, dst = edge_index[0], edge_index[1]

    deg_parts = _seg_sum(jnp.ones((N, 16), jnp.float32), src, dst)
    deg = deg_parts[:N, 0] + deg_parts[NPAD:NPAD + N, 0]
    dinv = lax.rsqrt(deg + 1.0)

    perm = jnp.argsort(dst)
    srt_pad = jnp.pad(jnp.take(src, perm), (0, _KE))
    dst_pad = jnp.pad(jnp.take(dst, perm), (0, _KE),
                      constant_values=-1000000)
    rp = jnp.cumsum(deg.astype(jnp.int32))
    rp_full = jnp.concatenate([jnp.zeros((1,), jnp.int32), rp])
    rp_pad = jnp.pad(rp_full, (0, 10256 - (N + 1)), mode="edge")

    h = x
    for W, b in ((W1, b1), (W2, b2), (W3, b3)):
        f = h.shape[1]
        y = h * dinv[:, None]
        yp = jnp.pad(y, ((0, 0), (0, 16 - f))) if f < 16 else y
        if f > 64:
            halves = []
            for j in range(0, f, 64):
                p = _seg_sum(yp[:, j:j + 64], src, dst)
                halves.append(p[:N] + p[NPAD:NPAD + N])
            seg = jnp.concatenate(halves, axis=1)
        else:
            parts = _seg_sum(yp, src, dst)
            seg = (parts[:N, :f] + parts[NPAD:NPAD + N, :f])
        agg = (seg + y) * dinv[:, None]
        h = _dense(agg, W, b)
        fo = h.shape[1]
        tables = [h[:, j:j + 128] for j in range(0, fo, 128)] if fo > 128 else [h]
        mp = _seg_max(tables, rp_pad, srt_pad, dst_pad)
        m = jnp.concatenate(
            [mp[j * NSEG:j * NSEG + N] for j in range(len(tables))], axis=1)
        h = jnp.maximum(m, h)
    return h


# speedup vs baseline: 14.2304x; 1.1076x over previous
import functools

import jax
import jax.numpy as jnp
from jax import lax
from jax.experimental import pallas as pl
from jax.experimental.pallas import tpu as pltpu
from jax.experimental.pallas import tpu_sc as plsc

N = 10000
E = 320000
NC, NS = 2, 16
NW = NC * NS
EPW = E // NW
ROWS_PER_SUB = 632
NPAD = ROWS_PER_SUB * NS

_MESH = plsc.VectorSubcoreMesh(core_axis_name="c", subcore_axis_name="s")


def _seg_sum(y, src, dst):
    F = y.shape[1]
    assert F <= 64
    K = 1000

    @functools.partial(
        pl.kernel,
        out_type=jax.ShapeDtypeStruct((NC * NPAD, F), jnp.float32),
        mesh=_MESH,
        compiler_params=pltpu.CompilerParams(use_tc_tiling_on_sc=False),
        scratch_types=[
            pltpu.VMEM((K,), jnp.int32),
            pltpu.VMEM((K,), jnp.int32),
            pltpu.VMEM((K, F), jnp.float32),
            pltpu.VMEM_SHARED((NPAD, F), jnp.float32),
            pltpu.SemaphoreType.DMA,
        ],
    )
    def ksum(y_hbm, src_hbm, dst_hbm, zero_hbm, out_hbm, sidx, didx, rows, acc, sem):
        cid = lax.axis_index("c")
        sid = lax.axis_index("s")
        wid = sid * NC + cid
        pltpu.sync_copy(
            zero_hbm.at[pl.ds(sid * ROWS_PER_SUB, ROWS_PER_SUB)],
            acc.at[pl.ds(sid * ROWS_PER_SUB, ROWS_PER_SUB)],
        )
        plsc.subcore_barrier()
        base = wid * EPW

        @pl.loop(0, EPW, step=K)
        def _(i):
            pltpu.sync_copy(src_hbm.at[pl.ds(base + i, K)], sidx)
            pltpu.sync_copy(dst_hbm.at[pl.ds(base + i, K)], didx)
            pltpu.async_copy(y_hbm.at[sidx], rows, sem).wait()
            pltpu.sync_copy(rows, acc.at[didx], add=True)

        plsc.subcore_barrier()
        pltpu.sync_copy(
            acc.at[pl.ds(sid * ROWS_PER_SUB, ROWS_PER_SUB)],
            out_hbm.at[pl.ds(cid * NPAD + sid * ROWS_PER_SUB, ROWS_PER_SUB)],
        )

    zeros = jnp.zeros((NPAD, F), jnp.float32)
    return ksum(y, src, dst, zeros)


NODES_PER_SUB = 320
NSEG = NODES_PER_SUB * NW
_KE = 512


def _seg_max(tables, rp_pad, srt_pad, dst_pad):
    nt = len(tables)
    FC = tables[0].shape[1]
    nfc = FC // 16
    KE = 256 if FC > 64 else 512
    RW = 336

    @functools.partial(
        pl.kernel,
        out_type=jax.ShapeDtypeStruct((nt * NSEG * FC,), jnp.float32),
        mesh=_MESH,
        compiler_params=pltpu.CompilerParams(use_tc_tiling_on_sc=False),
        scratch_types=[
            pltpu.VMEM((RW,), jnp.int32),
            pltpu.VMEM((KE,), jnp.int32),
            pltpu.VMEM((KE,), jnp.int32),
            pltpu.VMEM((KE,), jnp.int32),
            pltpu.VMEM((KE,), jnp.int32),
            pltpu.VMEM((KE, FC), jnp.float32),
            pltpu.VMEM((KE, FC), jnp.float32),
            pltpu.VMEM(((NODES_PER_SUB + 1) * FC,), jnp.float32),
            pltpu.SemaphoreType.DMA,
            pltpu.SemaphoreType.DMA,
        ],
    )
    def kmax(*refs):
        (rp_hbm, srt_hbm, dst_hbm), tabs, scratch = (
            refs[:3], refs[3:3 + nt], refs[3 + nt:])
        (out_hbm, rpv, sidx0, sidx1, dstv0, dstv1, rows0, rows1, obuf,
         sem0, sem1) = scratch
        cid = lax.axis_index("c")
        sid = lax.axis_index("s")
        wid = sid * NC + cid
        lo = wid * NODES_PER_SUB
        pltpu.sync_copy(rp_hbm.at[pl.ds(lo, RW)], rpv)
        S = rpv[pl.ds(0, 16)][0]
        T = rpv[pl.ds(NODES_PER_SUB, 16)][0]
        S16 = pl.multiple_of((S // 16) * 16, 8)
        T16 = ((T + 15) // 16) * 16
        nch = (T16 - S16 + KE - 1) // KE
        npair = (nch + 1) // 2
        zero = jnp.zeros((16,), jnp.float32)

        def rsafe(d):
            r = d - lo
            ok = (r >= 0) & (r < NODES_PER_SUB)
            return jnp.where(ok, r, NODES_PER_SUB)

        for t in range(nt):
            @pl.loop(0, NODES_PER_SUB + 1)
            def _(i):
                for k in range(nfc):
                    obuf[pl.ds(i * FC + 16 * k, 16)] = zero

            def fetch(c, sidx, dstv, rows, sem):
                lo_c = pl.multiple_of(S16 + c * KE, 8)
                pltpu.sync_copy(srt_hbm.at[pl.ds(lo_c, KE)], sidx)
                pltpu.sync_copy(dst_hbm.at[pl.ds(lo_c, KE)], dstv)
                pltpu.make_async_copy(tabs[t].at[sidx], rows, sem).start()

            def consume(c, st, dstv, rows, sem):
                lo_c = pl.multiple_of(S16 + c * KE, 8)
                ng = jnp.maximum(
                    (jnp.minimum(lo_c + KE, T16) - lo_c) // 16, 0)
                pltpu.make_async_copy(tabs[t].at[sidx0], rows, sem).wait()

                def do_group(gi, st):
                    base = gi * 16
                    dvec = dstv[pl.ds(base, 16)]
                    for l in range(16):
                        pd, acc = st[0], st[1:]
                        d = dvec[l]
                        neq = d != pd
                        rowk = tuple(rows[base + l, pl.ds(16 * k, 16)]
                                     for k in range(nfc))

                        def flush(pd=pd, acc=acc):
                            ro = rsafe(pd) * FC
                            for k in range(nfc):
                                obuf[pl.ds(ro + 16 * k, 16)] = acc[k]

                        lax.cond(neq, flush, lambda: None)
                        st = (d,) + tuple(
                            jnp.maximum(jnp.where(neq, zero, acc[k]), rowk[k])
                            for k in range(nfc))
                    return st

                return lax.fori_loop(0, ng, do_group, st)

            fetch(0, sidx0, dstv0, rows0, sem0)

            def do_pair(p, st):
                c0 = 2 * p
                fetch(c0 + 1, sidx1, dstv1, rows1, sem1)
                st = consume(c0, st, dstv0, rows0, sem0)
                fetch(c0 + 2, sidx0, dstv0, rows0, sem0)
                st = consume(c0 + 1, st, dstv1, rows1, sem1)
                return st

            st = lax.fori_loop(0, npair, do_pair,
                               (jnp.int32(-1),) + (zero,) * nfc)
            pltpu.make_async_copy(tabs[t].at[sidx0], rows0, sem0).wait()
            pd, acc = st[0], st[1:]
            ro = rsafe(pd) * FC
            for k in range(nfc):
                obuf[pl.ds(ro + 16 * k, 16)] = acc[k]
            pltpu.sync_copy(
                obuf.at[pl.ds(0, NODES_PER_SUB * FC)],
                out_hbm.at[pl.ds((t * NSEG + lo) * FC, NODES_PER_SUB * FC)])

    return kmax(rp_pad, srt_pad, dst_pad, *tables).reshape(nt * NSEG, FC)


def _dense_body(a_ref, w_ref, b_ref, o_ref):
    out = jnp.dot(a_ref[...], w_ref[...], preferred_element_type=jnp.float32)
    o_ref[...] = jax.nn.relu(out + b_ref[...])


def _dense(a, w, b):
    n, _ = a.shape
    d_out = w.shape[1]
    return pl.pallas_call(
        _dense_body,
        out_shape=jax.ShapeDtypeStruct((n, d_out), jnp.float32),
    )(a, w, b.reshape(1, d_out))


def kernel(x, edge_index, W1, b1, W2, b2, W3, b3):
    src, dst = edge_index[0], edge_index[1]

    deg_parts = _seg_sum(jnp.ones((N, 16), jnp.float32), src, dst)
    deg = deg_parts[:N, 0] + deg_parts[NPAD:NPAD + N, 0]
    dinv = lax.rsqrt(deg + 1.0)

    perm = jnp.argsort(dst)
    srt_pad = jnp.pad(jnp.take(src, perm), (0, 4 * _KE))
    dst_pad = jnp.pad(jnp.take(dst, perm), (0, 4 * _KE),
                      constant_values=-1000000)
    rp = jnp.cumsum(deg.astype(jnp.int32))
    rp_full = jnp.concatenate([jnp.zeros((1,), jnp.int32), rp])
    rp_pad = jnp.pad(rp_full, (0, 10256 - (N + 1)), mode="edge")

    h = x
    for W, b in ((W1, b1), (W2, b2), (W3, b3)):
        f = h.shape[1]
        y = h * dinv[:, None]
        yp = jnp.pad(y, ((0, 0), (0, 16 - f))) if f < 16 else y
        if f > 64:
            halves = []
            for j in range(0, f, 64):
                p = _seg_sum(yp[:, j:j + 64], src, dst)
                halves.append(p[:N] + p[NPAD:NPAD + N])
            seg = jnp.concatenate(halves, axis=1)
        else:
            parts = _seg_sum(yp, src, dst)
            seg = (parts[:N, :f] + parts[NPAD:NPAD + N, :f])
        agg = (seg + y) * dinv[:, None]
        h = _dense(agg, W, b)
        fo = h.shape[1]
        tables = [h[:, j:j + 128] for j in range(0, fo, 128)] if fo > 128 else [h]
        mp = _seg_max(tables, rp_pad, srt_pad, dst_pad)
        m = jnp.concatenate(
            [mp[j * NSEG:j * NSEG + N] for j in range(len(tables))], axis=1)
        h = jnp.maximum(m, h)
    return h
